# baseline (speedup 1.0000x reference)
import functools

import jax
import jax.numpy as jnp
from jax import lax
from jax.experimental import pallas as pl
from jax.experimental.pallas import tpu as pltpu
from jax.experimental.pallas import tpu_sc as plsc

DIM_X = 32
DIM_U = 8
NUM_P = 524288
U_SAMPLES = 8
ALPHA = 1.0
SIGMA_U = 10.0
SIG_DYN = 0.01

N_OUT = NUM_P // U_SAMPLES


_info = plsc.get_sparse_core_info()
_NC, _NS, _L = _info.num_cores, _info.num_subcores, _info.num_lanes
_NW = _NC * _NS
_CHUNK = N_OUT // _NW
_G = _CHUNK // 128

_sc_mesh = plsc.VectorSubcoreMesh(core_axis_name="c", subcore_axis_name="s")


def _i32x16(v):
    return jnp.full((16,), v, jnp.int32)


@functools.partial(
    pl.kernel,
    mesh=_sc_mesh,
    compiler_params=pltpu.CompilerParams(needs_layout_passes=False),
    out_type=[
        jax.ShapeDtypeStruct((N_OUT * DIM_X,), jnp.float32),
        jax.ShapeDtypeStruct((N_OUT * DIM_U,), jnp.float32),
        jax.ShapeDtypeStruct((N_OUT // 128, 128), jnp.int32),
    ],
    scratch_types=[
        pltpu.VMEM((_G, 128), jnp.int32),
        pltpu.VMEM((_G, 128), jnp.int32),
        pltpu.VMEM((_G, 128), jnp.int32),
        pltpu.VMEM((_G, 128), jnp.int32),
        pltpu.VMEM((128, 128), jnp.float32),
        pltpu.VMEM((128, 128), jnp.float32),
        pltpu.VMEM((_CHUNK * DIM_X,), jnp.float32),
        pltpu.VMEM((_CHUNK * DIM_U,), jnp.float32),
        pltpu.SemaphoreType.DMA,
        pltpu.SemaphoreType.DMA,
    ],
)
def _sc_gather(samples2d, p128, nu128,
               sel_x, sel_u, anc_out,
               idx_v, xrow_v, urow_v, anc_v, buf_x, buf_u, x_v, u_v,
               sem_x, sem_u):
    wid = lax.axis_index("s") * _NC + lax.axis_index("c")
    base = wid * _CHUNK
    iota16 = lax.iota(jnp.int32, 16)
    pltpu.sync_copy(samples2d.at[pl.ds(wid * _G, _G)], idx_v)
    for j in range(_G):
        for i in range(128 // _L):
            s = idx_v[j, pl.ds(i * _L, _L)]
            xrow_v[j, pl.ds(i * _L, _L)] = s >> 5
            urow_v[j, pl.ds(i * _L, _L)] = s >> 4
            anc_v[j, pl.ds(i * _L, _L)] = s >> 3
    for j in range(_G):
        cx = pltpu.async_copy(p128.at[xrow_v.at[j]], buf_x, sem_x)
        cu = pltpu.async_copy(nu128.at[urow_v.at[j]], buf_u, sem_u)
        cx.wait()
        cu.wait()

        @plsc.parallel_loop(0, 128 // _L)
        def _extract(t):
            rows = t * _L + iota16
            s = plsc.load_gather(idx_v, [_i32x16(j), rows])
            out_rows = j * 128 + rows
            colx0 = ((s >> 3) & 3) * DIM_X
            xflat0 = out_rows * DIM_X
            for c in range(DIM_X):
                val = plsc.load_gather(buf_x, [rows, colx0 + c])
                plsc.store_scatter(x_v, [xflat0 + c], val)
            colu0 = (s & 15) * DIM_U
            uflat0 = out_rows * DIM_U
            for c in range(DIM_U):
                val = plsc.load_gather(buf_u, [rows, colu0 + c])
                plsc.store_scatter(u_v, [uflat0 + c], val)

    pltpu.sync_copy(x_v, sel_x.at[pl.ds(base * DIM_X, _CHUNK * DIM_X)])
    pltpu.sync_copy(u_v, sel_u.at[pl.ds(base * DIM_U, _CHUNK * DIM_U)])
    pltpu.sync_copy(anc_v, anc_out.at[pl.ds(wid * _G, _G)])


_BLK = 4096


def _tc_dyn_body(x_ref, u_ref, n_ref, at_ref, bt_ref, q_ref, r_ref,
                 newp_ref, selp_ref, logw_ref):
    x = x_ref[...]
    u = u_ref[...]
    newp_ref[...] = (
        jnp.dot(x, at_ref[...], preferred_element_type=jnp.float32,
                precision=lax.Precision.HIGHEST)
        + jnp.dot(u, bt_ref[...], preferred_element_type=jnp.float32,
                  precision=lax.Precision.HIGHEST)
        + SIG_DYN * n_ref[...])
    selp_ref[:, :DIM_X] = x
    selp_ref[:, DIM_X:] = u
    logw_ref[...] = -ALPHA * (
        jnp.sum(x * x * q_ref[...], axis=1, keepdims=True)
        + jnp.sum(u * u * r_ref[...], axis=1, keepdims=True))


def _tc_dynamics(sel_x, sel_u, noise, A, B, Q_diag, R_diag):
    grid = (N_OUT // _BLK,)
    return pl.pallas_call(
        _tc_dyn_body,
        grid=grid,
        in_specs=[
            pl.BlockSpec((_BLK, DIM_X), lambda i: (i, 0)),
            pl.BlockSpec((_BLK, DIM_U), lambda i: (i, 0)),
            pl.BlockSpec((_BLK, DIM_X), lambda i: (i, 0)),
            pl.BlockSpec((DIM_X, DIM_X), lambda i: (0, 0)),
            pl.BlockSpec((DIM_U, DIM_X), lambda i: (0, 0)),
            pl.BlockSpec((1, DIM_X), lambda i: (0, 0)),
            pl.BlockSpec((1, DIM_U), lambda i: (0, 0)),
        ],
        out_specs=[
            pl.BlockSpec((_BLK, DIM_X), lambda i: (i, 0)),
            pl.BlockSpec((_BLK, DIM_X + DIM_U), lambda i: (i, 0)),
            pl.BlockSpec((_BLK, 1), lambda i: (i, 0)),
        ],
        out_shape=[
            jax.ShapeDtypeStruct((N_OUT, DIM_X), jnp.float32),
            jax.ShapeDtypeStruct((N_OUT, DIM_X + DIM_U), jnp.float32),
            jax.ShapeDtypeStruct((N_OUT, 1), jnp.float32),
        ],
    )(sel_x, sel_u, noise, A.T, B.T, Q_diag[None, :], R_diag[None, :])


def kernel(particles, iteration, K, b, A, B, Q_diag, R_diag):
    key = jax.random.fold_in(jax.random.key(42), iteration)
    k1, k2, k3 = jax.random.split(key, 3)
    n_out = N_OUT
    mean_u = particles @ K.T + b
    mean_u_rep = jnp.repeat(mean_u, U_SAMPLES, axis=0)
    eps = jax.random.normal(k1, mean_u_rep.shape, dtype=jnp.float32)
    new_u = mean_u_rep + SIGMA_U * eps
    part_rep = jnp.repeat(particles, U_SAMPLES, axis=0)
    cost = (jnp.sum(part_rep * part_rep * Q_diag[None, :], axis=1)
            + jnp.sum(new_u * new_u * R_diag[None, :], axis=1))
    log_w = -ALPHA * cost
    logits = log_w - jax.scipy.special.logsumexp(log_w)
    u01 = jax.random.uniform(k2, logits.shape, dtype=jnp.float32)
    gumbel = -jnp.log(-jnp.log(u01 + 1e-20) + 1e-20)
    _, samples = jax.lax.top_k(jax.lax.stop_gradient(logits) + gumbel, n_out)
    dyn_noise = jax.random.normal(k3, (n_out, DIM_X), dtype=jnp.float32)

    samples2d = samples.reshape(N_OUT // 128, 128)
    p128 = particles.reshape(N_OUT * DIM_X // 128, 128)
    nu128 = new_u.reshape(NUM_P * DIM_U // 128, 128)
    sel_xf, sel_uf, anc2d = _sc_gather(samples2d, p128, nu128)
    sel_x = sel_xf.reshape(N_OUT, DIM_X)
    sel_u = sel_uf.reshape(N_OUT, DIM_U)
    ancestors = anc2d.reshape(N_OUT)
    new_particles, sel_particles, logw2d = _tc_dynamics(
        sel_x, sel_u, dyn_noise, A, B, Q_diag, R_diag)
    return (new_particles, sel_particles, logw2d.reshape(N_OUT), ancestors)


# speedup vs baseline: 1.3545x; 1.3545x over previous
import functools

import jax
import jax.numpy as jnp
from jax import lax
from jax.experimental import pallas as pl
from jax.experimental.pallas import tpu as pltpu
from jax.experimental.pallas import tpu_sc as plsc

DIM_X = 32
DIM_U = 8
NUM_P = 524288
U_SAMPLES = 8
ALPHA = 1.0
SIGMA_U = 10.0
SIG_DYN = 0.01

N_OUT = NUM_P // U_SAMPLES


_info = plsc.get_sparse_core_info()
_NC, _NS, _L = _info.num_cores, _info.num_subcores, _info.num_lanes
_NW = _NC * _NS
_CHUNK = N_OUT // _NW
_G = _CHUNK // 128

_sc_mesh = plsc.VectorSubcoreMesh(core_axis_name="c", subcore_axis_name="s")


def _i32x16(v):
    return jnp.full((16,), v, jnp.int32)


@functools.partial(
    pl.kernel,
    mesh=_sc_mesh,
    compiler_params=pltpu.CompilerParams(needs_layout_passes=False),
    out_type=[
        jax.ShapeDtypeStruct((N_OUT * DIM_X,), jnp.float32),
        jax.ShapeDtypeStruct((N_OUT * DIM_U,), jnp.float32),
        jax.ShapeDtypeStruct((N_OUT // 128, 128), jnp.int32),
    ],
    scratch_types=[
        pltpu.VMEM((_G, 128), jnp.int32),
        pltpu.VMEM((_G, 128), jnp.int32),
        pltpu.VMEM((_G, 128), jnp.int32),
        pltpu.VMEM((_G, 128), jnp.int32),
        pltpu.VMEM((128, 128), jnp.float32),
        pltpu.VMEM((128, 128), jnp.float32),
        pltpu.VMEM((_CHUNK * DIM_X,), jnp.float32),
        pltpu.VMEM((_CHUNK * DIM_U,), jnp.float32),
        pltpu.SemaphoreType.DMA,
        pltpu.SemaphoreType.DMA,
    ],
)
def _sc_gather(samples2d, p128, nu128,
               sel_x, sel_u, anc_out,
               idx_v, xrow_v, urow_v, anc_v, buf_x, buf_u, x_v, u_v,
               sem_x, sem_u):
    wid = lax.axis_index("s") * _NC + lax.axis_index("c")
    base = wid * _CHUNK
    iota16 = lax.iota(jnp.int32, 16)
    pltpu.sync_copy(samples2d.at[pl.ds(wid * _G, _G)], idx_v)
    for j in range(_G):
        for i in range(128 // _L):
            s = idx_v[j, pl.ds(i * _L, _L)]
            xrow_v[j, pl.ds(i * _L, _L)] = s >> 5
            urow_v[j, pl.ds(i * _L, _L)] = s >> 4
            anc_v[j, pl.ds(i * _L, _L)] = s >> 3
    for j in range(_G):
        cx = pltpu.async_copy(p128.at[xrow_v.at[j]], buf_x, sem_x)
        cu = pltpu.async_copy(nu128.at[urow_v.at[j]], buf_u, sem_u)
        cx.wait()
        cu.wait()

        @plsc.parallel_loop(0, 128 // _L)
        def _extract(t):
            rows = t * _L + iota16
            s = plsc.load_gather(idx_v, [_i32x16(j), rows])
            out_rows = j * 128 + rows
            colx0 = ((s >> 3) & 3) * DIM_X
            xflat0 = out_rows * DIM_X
            for c in range(DIM_X):
                val = plsc.load_gather(buf_x, [rows, colx0 + c])
                plsc.store_scatter(x_v, [xflat0 + c], val)
            colu0 = (s & 15) * DIM_U
            uflat0 = out_rows * DIM_U
            for c in range(DIM_U):
                val = plsc.load_gather(buf_u, [rows, colu0 + c])
                plsc.store_scatter(u_v, [uflat0 + c], val)

    pltpu.sync_copy(x_v, sel_x.at[pl.ds(base * DIM_X, _CHUNK * DIM_X)])
    pltpu.sync_copy(u_v, sel_u.at[pl.ds(base * DIM_U, _CHUNK * DIM_U)])
    pltpu.sync_copy(anc_v, anc_out.at[pl.ds(wid * _G, _G)])


N_KEYS = NUM_P
_SW = 16
_SHARD = N_KEYS // _SW
_KB = 4096
_MMAX = 73728
_TRASH = 64
_SLICE = _MMAX // _SW
_NB = 2048
_K_TARGET = N_OUT - 1

_sort_mesh = plsc.VectorSubcoreMesh(
    core_axis_name="c", subcore_axis_name="s", num_cores=1)


def _srl(x, n):
    return lax.shift_right_logical(x, n)


@functools.partial(
    pl.kernel,
    mesh=_sort_mesh,
    compiler_params=pltpu.CompilerParams(needs_layout_passes=False),
    out_type=jax.ShapeDtypeStruct((N_OUT,), jnp.int32),
    scratch_types=[
        pltpu.VMEM((_KB // 128, 128), jnp.float32),
        pltpu.VMEM((_NB * 16,), jnp.int32),
        pltpu.VMEM((_SW, _NB), jnp.int32),
        pltpu.VMEM((1, _NB), jnp.int32),
        pltpu.VMEM((_SLICE,), jnp.int32),
        pltpu.VMEM((_SLICE,), jnp.int32),
        pltpu.VMEM((1, 128), jnp.int32),
        pltpu.VMEM((128,), jnp.int32),
        pltpu.VMEM((128,), jnp.int32),
        pltpu.VMEM_SHARED((_SW, _NB), jnp.int32),
        pltpu.VMEM_SHARED((_MMAX + _TRASH,), jnp.int32),
        pltpu.VMEM_SHARED((_MMAX + _TRASH,), jnp.int32),
        pltpu.VMEM_SHARED((_MMAX + _TRASH,), jnp.int32),
        pltpu.VMEM_SHARED((_MMAX + _TRASH,), jnp.int32),
        pltpu.SemaphoreType.DMA,
    ],
)
def _sc_topk_sort(keys2d, samples_out,
                  kbuf, hist_v, grid_v, ctr_v, oslice, islice,
                  st_pos, st_ord, st_idx,
                  gsp, sao, sai, sbo, sbi, sem):
    w = lax.axis_index("s")
    iota16 = lax.iota(jnp.int32, 16)
    ones16 = jnp.full((16,), 1, jnp.int32)
    zeros16 = jnp.full((16,), 0, jnp.int32)

    def w16(v):
        return jnp.full((16,), v, jnp.int32)

    @plsc.parallel_loop(0, _NB)
    def _z(i):
        plsc.store_scatter(hist_v, [i * 16 + iota16], zeros16)

    def _key_sweep(consume, init):
        carry = init
        for blk in range(_SHARD // _KB):
            pltpu.sync_copy(
                keys2d.at[pl.ds(w * (_SHARD // 128) + blk * (_KB // 128),
                                _KB // 128)],
                kbuf)

            def body(i, c):
                fl = i * 16 + iota16
                k = plsc.load_gather(kbuf, [_srl(fl, 7), fl & 127])
                s = plsc.bitcast(k, jnp.int32)
                u = jnp.where(s < 0, s, ~s & jnp.int32(0x7FFFFFFF))
                return consume(u, blk * _KB + fl, c)
            carry = lax.fori_loop(0, _KB // 16, body, carry)
        return carry

    def _hist_consume(digit_fn, mask_fn):
        def consume(u, fl, c):
            d = digit_fn(u)
            plsc.addupdate_scatter(hist_v, [d * 16 + iota16], ones16,
                                   mask=mask_fn(u))
            return c
        return consume

    _key_sweep(_hist_consume(lambda u: _srl(u, 21), lambda u: None),
               jnp.int32(0))

    def _reduce_hist_to_row_and_publish():
        def body(c, carry):
            acc = zeros16
            for l in range(16):
                idx = (c * 16 + iota16) * 16 + l
                acc = acc + plsc.load_gather(hist_v, [idx])
                plsc.store_scatter(hist_v, [idx], zeros16)
            plsc.store_scatter(ctr_v, [zeros16, c * 16 + iota16], acc)
            return carry
        lax.fori_loop(0, _NB // 16, body, jnp.int32(0))
        pltpu.sync_copy(ctr_v, gsp.at[pl.ds(w, 1)])

    _reduce_hist_to_row_and_publish()
    plsc.subcore_barrier()

    pltpu.sync_copy(gsp, grid_v)
    plsc.subcore_barrier()

    def _find_cut(target):
        def body(c, carry):
            run, cut, rank = carry
            tot = zeros16
            for v in range(_SW):
                tot = tot + plsc.load_gather(
                    grid_v, [w16(v), c * 16 + iota16])
            cs = plsc.cumsum(tot)
            incl = cs + run
            excl = incl - tot
            cond = (excl <= target) & (incl > target)
            binidx = c * 16 + iota16
            cut = cut + jnp.sum(jnp.where(cond, binidx, 0))
            rank = rank + jnp.sum(jnp.where(cond, excl, 0))
            return run + jnp.sum(tot), cut, rank
        _, cut, rank = lax.fori_loop(
            0, _NB // 16, body,
            (jnp.int32(0), jnp.int32(0), jnp.int32(0)))
        return cut, rank

    cut1, rank1 = _find_cut(jnp.int32(_K_TARGET))

    def _pref_body(b, acc):
        v16 = plsc.load_gather(grid_v, [iota16, w16(b)])
        return acc + jnp.where(w16(b < cut1) > 0, v16, 0)
    pref1 = lax.fori_loop(0, _NB, _pref_body, zeros16)

    _key_sweep(_hist_consume(lambda u: _srl(u, 10) & jnp.int32(0x7FF),
                             lambda u: _srl(u, 21) == cut1),
               jnp.int32(0))
    _reduce_hist_to_row_and_publish()
    plsc.subcore_barrier()

    pltpu.sync_copy(gsp, grid_v)
    cut2, _ = _find_cut(_K_TARGET - rank1)

    def _pref2_body(b, acc):
        v16 = plsc.load_gather(grid_v, [iota16, w16(b)])
        return acc + jnp.where(w16(b <= cut2) > 0, v16, 0)
    cnt16 = pref1 + lax.fori_loop(0, _NB, _pref2_body, zeros16)
    off16 = plsc.cumsum(cnt16) - cnt16
    my_off = jnp.sum(jnp.where(iota16 == w, off16, 0))

    for k in range(8):
        st_ord[pl.ds(k * 16, 16)] = jnp.full((16,), -1, jnp.int32)
    for b in range(_SLICE // 128):
        pltpu.sync_copy(st_ord, sao.at[pl.ds(w * _SLICE + b * 128, 128)])
    plsc.subcore_barrier()

    cuthi = (cut1 << 11) | cut2

    run = my_off
    for blk in range(_SHARD // _KB):
        pltpu.sync_copy(
            keys2d.at[pl.ds(w * (_SHARD // 128) + blk * (_KB // 128),
                            _KB // 128)],
            kbuf)

        def _compact_batch(b, r):
            for k in range(8):
                fl = b * 128 + k * 16 + iota16
                kv = plsc.load_gather(kbuf, [_srl(fl, 7), fl & 127])
                s = plsc.bitcast(kv, jnp.int32)
                u = jnp.where(s < 0, s, ~s & jnp.int32(0x7FFFFFFF))
                m = (_srl(u, 10) <= cuthi)
                mi = jnp.where(m, 1, 0).astype(jnp.int32)
                cs = plsc.cumsum(mi)
                pos = jnp.where(m, r + cs - 1, _MMAX + iota16)
                pos = jnp.minimum(pos, _MMAX + _TRASH - 1)
                st_pos[0, pl.ds(k * 16, 16)] = pos
                st_ord[pl.ds(k * 16, 16)] = u
                st_idx[pl.ds(k * 16, 16)] = w * _SHARD + blk * _KB + fl
                r = r + jnp.sum(mi)
            pltpu.sync_copy(st_ord, sao.at[st_pos.at[0]])
            pltpu.sync_copy(st_idx, sai.at[st_pos.at[0]])
            return r
        run = lax.fori_loop(0, _KB // 128, _compact_batch, run)
    plsc.subcore_barrier()

    def _radix_pass(srco, srci, dsto, dsti, digit_fn):
        pltpu.sync_copy(srco.at[pl.ds(w * _SLICE, _SLICE)], oslice)
        pltpu.sync_copy(srci.at[pl.ds(w * _SLICE, _SLICE)], islice)

        def hbody(i, carry):
            u = plsc.load_gather(oslice, [i * 16 + iota16])
            d = digit_fn(u)
            plsc.addupdate_scatter(hist_v, [d * 16 + iota16], ones16)
            return carry
        lax.fori_loop(0, _SLICE // 16, hbody, jnp.int32(0))
        _reduce_hist_to_row_and_publish()
        plsc.subcore_barrier()

        pltpu.sync_copy(gsp, grid_v)
        plsc.subcore_barrier()

        def obody(c, run):
            tot = zeros16
            sub = zeros16
            for v in range(_SW):
                g = plsc.load_gather(grid_v, [w16(v), c * 16 + iota16])
                tot = tot + g
                sub = sub + jnp.where(w16(v < w) > 0, g, 0)
            cs = plsc.cumsum(tot)
            excl = cs - tot + run
            plsc.store_scatter(ctr_v, [zeros16, c * 16 + iota16], excl + sub)
            return run + jnp.sum(tot)
        lax.fori_loop(0, _NB // 16, obody, jnp.int32(0))

        def pbody(b, carry):
            for k in range(8):
                ls = b * 128 + k * 16 + iota16
                u = plsc.load_gather(oslice, [ls])
                ix = plsc.load_gather(islice, [ls])
                d = digit_fn(u)
                bs = plsc.load_gather(ctr_v, [zeros16, d])
                cnt, last = plsc.scan_count(d)
                pos = bs + cnt - 1
                plsc.store_scatter(ctr_v, [zeros16, d], bs + cnt, mask=last)
                st_pos[0, pl.ds(k * 16, 16)] = pos
                st_ord[pl.ds(k * 16, 16)] = u
                st_idx[pl.ds(k * 16, 16)] = ix
            pltpu.sync_copy(st_ord, dsto.at[st_pos.at[0]])
            pltpu.sync_copy(st_idx, dsti.at[st_pos.at[0]])
            return carry
        lax.fori_loop(0, _SLICE // 128, pbody, jnp.int32(0))
        plsc.subcore_barrier()

    _radix_pass(sao, sai, sbo, sbi, lambda u: u & jnp.int32(0x7FF))
    _radix_pass(sbo, sbi, sao, sai,
                lambda u: _srl(u, 11) & jnp.int32(0x7FF))
    _radix_pass(sao, sai, sbo, sbi, lambda u: _srl(u, 22))

    pltpu.sync_copy(
        sbi.at[pl.ds(w * (N_OUT // _SW), N_OUT // _SW)],
        samples_out.at[pl.ds(w * (N_OUT // _SW), N_OUT // _SW)])


_BLK = 4096


def _tc_dyn_body(x_ref, u_ref, n_ref, at_ref, bt_ref, q_ref, r_ref,
                 newp_ref, selp_ref, logw_ref):
    x = x_ref[...]
    u = u_ref[...]
    newp_ref[...] = (
        jnp.dot(x, at_ref[...], preferred_element_type=jnp.float32,
                precision=lax.Precision.HIGHEST)
        + jnp.dot(u, bt_ref[...], preferred_element_type=jnp.float32,
                  precision=lax.Precision.HIGHEST)
        + SIG_DYN * n_ref[...])
    selp_ref[:, :DIM_X] = x
    selp_ref[:, DIM_X:] = u
    logw_ref[...] = -ALPHA * (
        jnp.sum(x * x * q_ref[...], axis=1, keepdims=True)
        + jnp.sum(u * u * r_ref[...], axis=1, keepdims=True))


def _tc_dynamics(sel_x, sel_u, noise, A, B, Q_diag, R_diag):
    grid = (N_OUT // _BLK,)
    return pl.pallas_call(
        _tc_dyn_body,
        grid=grid,
        in_specs=[
            pl.BlockSpec((_BLK, DIM_X), lambda i: (i, 0)),
            pl.BlockSpec((_BLK, DIM_U), lambda i: (i, 0)),
            pl.BlockSpec((_BLK, DIM_X), lambda i: (i, 0)),
            pl.BlockSpec((DIM_X, DIM_X), lambda i: (0, 0)),
            pl.BlockSpec((DIM_U, DIM_X), lambda i: (0, 0)),
            pl.BlockSpec((1, DIM_X), lambda i: (0, 0)),
            pl.BlockSpec((1, DIM_U), lambda i: (0, 0)),
        ],
        out_specs=[
            pl.BlockSpec((_BLK, DIM_X), lambda i: (i, 0)),
            pl.BlockSpec((_BLK, DIM_X + DIM_U), lambda i: (i, 0)),
            pl.BlockSpec((_BLK, 1), lambda i: (i, 0)),
        ],
        out_shape=[
            jax.ShapeDtypeStruct((N_OUT, DIM_X), jnp.float32),
            jax.ShapeDtypeStruct((N_OUT, DIM_X + DIM_U), jnp.float32),
            jax.ShapeDtypeStruct((N_OUT, 1), jnp.float32),
        ],
    )(sel_x, sel_u, noise, A.T, B.T, Q_diag[None, :], R_diag[None, :])


def kernel(particles, iteration, K, b, A, B, Q_diag, R_diag):
    key = jax.random.fold_in(jax.random.key(42), iteration)
    k1, k2, k3 = jax.random.split(key, 3)
    n_out = N_OUT
    mean_u = particles @ K.T + b
    mean_u_rep = jnp.repeat(mean_u, U_SAMPLES, axis=0)
    eps = jax.random.normal(k1, mean_u_rep.shape, dtype=jnp.float32)
    new_u = mean_u_rep + SIGMA_U * eps
    part_rep = jnp.repeat(particles, U_SAMPLES, axis=0)
    cost = (jnp.sum(part_rep * part_rep * Q_diag[None, :], axis=1)
            + jnp.sum(new_u * new_u * R_diag[None, :], axis=1))
    log_w = -ALPHA * cost
    logits = log_w - jax.scipy.special.logsumexp(log_w)
    u01 = jax.random.uniform(k2, logits.shape, dtype=jnp.float32)
    gumbel = -jnp.log(-jnp.log(u01 + 1e-20) + 1e-20)
    keys = jax.lax.stop_gradient(logits) + gumbel
    samples = _sc_topk_sort(keys.reshape(N_KEYS // 128, 128))
    dyn_noise = jax.random.normal(k3, (n_out, DIM_X), dtype=jnp.float32)

    samples2d = samples.reshape(N_OUT // 128, 128)
    p128 = particles.reshape(N_OUT * DIM_X // 128, 128)
    nu128 = new_u.reshape(NUM_P * DIM_U // 128, 128)
    sel_xf, sel_uf, anc2d = _sc_gather(samples2d, p128, nu128)
    sel_x = sel_xf.reshape(N_OUT, DIM_X)
    sel_u = sel_uf.reshape(N_OUT, DIM_U)
    ancestors = anc2d.reshape(N_OUT)
    new_particles, sel_particles, logw2d = _tc_dynamics(
        sel_x, sel_u, dyn_noise, A, B, Q_diag, R_diag)
    return (new_particles, sel_particles, logw2d.reshape(N_OUT), ancestors)


# speedup vs baseline: 1.3546x; 1.0001x over previous
import functools

import jax
import jax.numpy as jnp
from jax import lax
from jax.experimental import pallas as pl
from jax.experimental.pallas import tpu as pltpu
from jax.experimental.pallas import tpu_sc as plsc

DIM_X = 32
DIM_U = 8
NUM_P = 524288
U_SAMPLES = 8
ALPHA = 1.0
SIGMA_U = 10.0
SIG_DYN = 0.01

N_OUT = NUM_P // U_SAMPLES


_info = plsc.get_sparse_core_info()
_NC, _NS, _L = _info.num_cores, _info.num_subcores, _info.num_lanes
_NW = _NC * _NS
_CHUNK = N_OUT // _NW
_G = _CHUNK // 128

_sc_mesh = plsc.VectorSubcoreMesh(core_axis_name="c", subcore_axis_name="s")


def _i32x16(v):
    return jnp.full((16,), v, jnp.int32)


@functools.partial(
    pl.kernel,
    mesh=_sc_mesh,
    compiler_params=pltpu.CompilerParams(needs_layout_passes=False),
    out_type=[
        jax.ShapeDtypeStruct((N_OUT * DIM_X,), jnp.float32),
        jax.ShapeDtypeStruct((N_OUT * DIM_U,), jnp.float32),
        jax.ShapeDtypeStruct((N_OUT // 128, 128), jnp.int32),
    ],
    scratch_types=[
        pltpu.VMEM((_G, 128), jnp.int32),
        pltpu.VMEM((_G, 128), jnp.int32),
        pltpu.VMEM((_G, 128), jnp.int32),
        pltpu.VMEM((_G, 128), jnp.int32),
        pltpu.VMEM((128, 128), jnp.float32),
        pltpu.VMEM((128, 128), jnp.float32),
        pltpu.VMEM((_CHUNK * DIM_X,), jnp.float32),
        pltpu.VMEM((_CHUNK * DIM_U,), jnp.float32),
        pltpu.SemaphoreType.DMA,
        pltpu.SemaphoreType.DMA,
    ],
)
def _sc_gather(samples2d, p128, nu128,
               sel_x, sel_u, anc_out,
               idx_v, xrow_v, urow_v, anc_v, buf_x, buf_u, x_v, u_v,
               sem_x, sem_u):
    wid = lax.axis_index("s") * _NC + lax.axis_index("c")
    base = wid * _CHUNK
    iota16 = lax.iota(jnp.int32, 16)
    pltpu.sync_copy(samples2d.at[pl.ds(wid * _G, _G)], idx_v)
    for j in range(_G):
        for i in range(128 // _L):
            s = idx_v[j, pl.ds(i * _L, _L)]
            xrow_v[j, pl.ds(i * _L, _L)] = s >> 5
            urow_v[j, pl.ds(i * _L, _L)] = s >> 4
            anc_v[j, pl.ds(i * _L, _L)] = s >> 3
    for j in range(_G):
        cx = pltpu.async_copy(p128.at[xrow_v.at[j]], buf_x, sem_x)
        cu = pltpu.async_copy(nu128.at[urow_v.at[j]], buf_u, sem_u)
        cx.wait()
        cu.wait()

        @plsc.parallel_loop(0, 128 // _L)
        def _extract(t):
            rows = t * _L + iota16
            s = plsc.load_gather(idx_v, [_i32x16(j), rows])
            out_rows = j * 128 + rows
            colx0 = ((s >> 3) & 3) * DIM_X
            xflat0 = out_rows * DIM_X
            for c in range(DIM_X):
                val = plsc.load_gather(buf_x, [rows, colx0 + c])
                plsc.store_scatter(x_v, [xflat0 + c], val)
            colu0 = (s & 15) * DIM_U
            uflat0 = out_rows * DIM_U
            for c in range(DIM_U):
                val = plsc.load_gather(buf_u, [rows, colu0 + c])
                plsc.store_scatter(u_v, [uflat0 + c], val)

    pltpu.sync_copy(x_v, sel_x.at[pl.ds(base * DIM_X, _CHUNK * DIM_X)])
    pltpu.sync_copy(u_v, sel_u.at[pl.ds(base * DIM_U, _CHUNK * DIM_U)])
    pltpu.sync_copy(anc_v, anc_out.at[pl.ds(wid * _G, _G)])


N_KEYS = NUM_P
_SW = 16
_SHARD = N_KEYS // _SW
_KB = 4096
_MMAX = 73728
_TRASH = 64
_SLICE = _MMAX // _SW
_NB = 2048
_K_TARGET = N_OUT - 1

_sort_mesh = plsc.VectorSubcoreMesh(
    core_axis_name="c", subcore_axis_name="s", num_cores=1)


def _srl(x, n):
    return lax.shift_right_logical(x, n)


@functools.partial(
    pl.kernel,
    mesh=_sort_mesh,
    compiler_params=pltpu.CompilerParams(needs_layout_passes=False),
    out_type=jax.ShapeDtypeStruct((N_OUT,), jnp.int32),
    scratch_types=[
        pltpu.VMEM((_KB // 128, 128), jnp.float32),
        pltpu.VMEM((_NB * 16,), jnp.int32),
        pltpu.VMEM((_SW, _NB), jnp.int32),
        pltpu.VMEM((1, _NB), jnp.int32),
        pltpu.VMEM((_SLICE,), jnp.int32),
        pltpu.VMEM((_SLICE,), jnp.int32),
        pltpu.VMEM((1, 128), jnp.int32),
        pltpu.VMEM((128,), jnp.int32),
        pltpu.VMEM((128,), jnp.int32),
        pltpu.VMEM_SHARED((_SW, _NB), jnp.int32),
        pltpu.VMEM_SHARED((_MMAX + _TRASH,), jnp.int32),
        pltpu.VMEM_SHARED((_MMAX + _TRASH,), jnp.int32),
        pltpu.VMEM_SHARED((_MMAX + _TRASH,), jnp.int32),
        pltpu.VMEM_SHARED((_MMAX + _TRASH,), jnp.int32),
        pltpu.SemaphoreType.DMA,
    ],
)
def _sc_topk_sort(keys2d, samples_out,
                  kbuf, hist_v, grid_v, ctr_v, oslice, islice,
                  st_pos, st_ord, st_idx,
                  gsp, sao, sai, sbo, sbi, sem):
    w = lax.axis_index("s")
    iota16 = lax.iota(jnp.int32, 16)
    ones16 = jnp.full((16,), 1, jnp.int32)
    zeros16 = jnp.full((16,), 0, jnp.int32)

    def w16(v):
        return jnp.full((16,), v, jnp.int32)

    @plsc.parallel_loop(0, _NB, unroll=8)
    def _z(i):
        plsc.store_scatter(hist_v, [i * 16 + iota16], zeros16)

    def _key_sweep(consume, init):
        carry = init
        for blk in range(_SHARD // _KB):
            pltpu.sync_copy(
                keys2d.at[pl.ds(w * (_SHARD // 128) + blk * (_KB // 128),
                                _KB // 128)],
                kbuf)

            def body(i, c):
                fl = i * 16 + iota16
                k = plsc.load_gather(kbuf, [_srl(fl, 7), fl & 127])
                s = plsc.bitcast(k, jnp.int32)
                u = jnp.where(s < 0, s, ~s & jnp.int32(0x7FFFFFFF))
                return consume(u, blk * _KB + fl, c)
            carry = lax.fori_loop(0, _KB // 16, body, carry, unroll=8)
        return carry

    def _hist_consume(digit_fn, mask_fn):
        def consume(u, fl, c):
            d = digit_fn(u)
            plsc.addupdate_scatter(hist_v, [d * 16 + iota16], ones16,
                                   mask=mask_fn(u))
            return c
        return consume

    _key_sweep(_hist_consume(lambda u: _srl(u, 21), lambda u: None),
               jnp.int32(0))

    def _reduce_hist_to_row_and_publish():
        def body(c, carry):
            acc = zeros16
            for l in range(16):
                idx = (c * 16 + iota16) * 16 + l
                acc = acc + plsc.load_gather(hist_v, [idx])
                plsc.store_scatter(hist_v, [idx], zeros16)
            plsc.store_scatter(ctr_v, [zeros16, c * 16 + iota16], acc)
            return carry
        lax.fori_loop(0, _NB // 16, body, jnp.int32(0))
        pltpu.sync_copy(ctr_v, gsp.at[pl.ds(w, 1)])

    _reduce_hist_to_row_and_publish()
    plsc.subcore_barrier()

    pltpu.sync_copy(gsp, grid_v)
    plsc.subcore_barrier()

    def _find_cut(target):
        def body(c, carry):
            run, cut, rank = carry
            tot = zeros16
            for v in range(_SW):
                tot = tot + plsc.load_gather(
                    grid_v, [w16(v), c * 16 + iota16])
            cs = plsc.cumsum(tot)
            incl = cs + run
            excl = incl - tot
            cond = (excl <= target) & (incl > target)
            binidx = c * 16 + iota16
            cut = cut + jnp.sum(jnp.where(cond, binidx, 0))
            rank = rank + jnp.sum(jnp.where(cond, excl, 0))
            return run + jnp.sum(tot), cut, rank
        _, cut, rank = lax.fori_loop(
            0, _NB // 16, body,
            (jnp.int32(0), jnp.int32(0), jnp.int32(0)))
        return cut, rank

    cut1, rank1 = _find_cut(jnp.int32(_K_TARGET))

    def _pref_body(b, acc):
        v16 = plsc.load_gather(grid_v, [iota16, w16(b)])
        return acc + jnp.where(w16(b < cut1) > 0, v16, 0)
    pref1 = lax.fori_loop(0, _NB, _pref_body, zeros16, unroll=16)

    _key_sweep(_hist_consume(lambda u: _srl(u, 10) & jnp.int32(0x7FF),
                             lambda u: _srl(u, 21) == cut1),
               jnp.int32(0))
    _reduce_hist_to_row_and_publish()
    plsc.subcore_barrier()

    pltpu.sync_copy(gsp, grid_v)
    cut2, _ = _find_cut(_K_TARGET - rank1)

    def _pref2_body(b, acc):
        v16 = plsc.load_gather(grid_v, [iota16, w16(b)])
        return acc + jnp.where(w16(b <= cut2) > 0, v16, 0)
    cnt16 = pref1 + lax.fori_loop(0, _NB, _pref2_body, zeros16, unroll=16)
    off16 = plsc.cumsum(cnt16) - cnt16
    my_off = jnp.sum(jnp.where(iota16 == w, off16, 0))

    for k in range(8):
        st_ord[pl.ds(k * 16, 16)] = jnp.full((16,), -1, jnp.int32)
    for b in range(_SLICE // 128):
        pltpu.sync_copy(st_ord, sao.at[pl.ds(w * _SLICE + b * 128, 128)])
    plsc.subcore_barrier()

    cuthi = (cut1 << 11) | cut2

    run = my_off
    for blk in range(_SHARD // _KB):
        pltpu.sync_copy(
            keys2d.at[pl.ds(w * (_SHARD // 128) + blk * (_KB // 128),
                            _KB // 128)],
            kbuf)

        def _compact_batch(b, r):
            for k in range(8):
                fl = b * 128 + k * 16 + iota16
                kv = plsc.load_gather(kbuf, [_srl(fl, 7), fl & 127])
                s = plsc.bitcast(kv, jnp.int32)
                u = jnp.where(s < 0, s, ~s & jnp.int32(0x7FFFFFFF))
                m = (_srl(u, 10) <= cuthi)
                mi = jnp.where(m, 1, 0).astype(jnp.int32)
                cs = plsc.cumsum(mi)
                pos = jnp.where(m, r + cs - 1, _MMAX + iota16)
                pos = jnp.minimum(pos, _MMAX + _TRASH - 1)
                st_pos[0, pl.ds(k * 16, 16)] = pos
                st_ord[pl.ds(k * 16, 16)] = u
                st_idx[pl.ds(k * 16, 16)] = w * _SHARD + blk * _KB + fl
                r = r + jnp.sum(mi)
            pltpu.sync_copy(st_ord, sao.at[st_pos.at[0]])
            pltpu.sync_copy(st_idx, sai.at[st_pos.at[0]])
            return r
        run = lax.fori_loop(0, _KB // 128, _compact_batch, run)
    plsc.subcore_barrier()

    def _radix_pass(srco, srci, dsto, dsti, digit_fn):
        pltpu.sync_copy(srco.at[pl.ds(w * _SLICE, _SLICE)], oslice)
        pltpu.sync_copy(srci.at[pl.ds(w * _SLICE, _SLICE)], islice)

        def hbody(i, carry):
            u = plsc.load_gather(oslice, [i * 16 + iota16])
            d = digit_fn(u)
            plsc.addupdate_scatter(hist_v, [d * 16 + iota16], ones16)
            return carry
        lax.fori_loop(0, _SLICE // 16, hbody, jnp.int32(0), unroll=8)
        _reduce_hist_to_row_and_publish()
        plsc.subcore_barrier()

        pltpu.sync_copy(gsp, grid_v)
        plsc.subcore_barrier()

        def obody(c, run):
            tot = zeros16
            sub = zeros16
            for v in range(_SW):
                g = plsc.load_gather(grid_v, [w16(v), c * 16 + iota16])
                tot = tot + g
                sub = sub + jnp.where(w16(v < w) > 0, g, 0)
            cs = plsc.cumsum(tot)
            excl = cs - tot + run
            plsc.store_scatter(ctr_v, [zeros16, c * 16 + iota16], excl + sub)
            return run + jnp.sum(tot)
        lax.fori_loop(0, _NB // 16, obody, jnp.int32(0))

        def pbody(b, carry):
            for k in range(8):
                ls = b * 128 + k * 16 + iota16
                u = plsc.load_gather(oslice, [ls])
                ix = plsc.load_gather(islice, [ls])
                d = digit_fn(u)
                bs = plsc.load_gather(ctr_v, [zeros16, d])
                cnt, last = plsc.scan_count(d)
                pos = bs + cnt - 1
                plsc.store_scatter(ctr_v, [zeros16, d], bs + cnt, mask=last)
                st_pos[0, pl.ds(k * 16, 16)] = pos
                st_ord[pl.ds(k * 16, 16)] = u
                st_idx[pl.ds(k * 16, 16)] = ix
            pltpu.sync_copy(st_ord, dsto.at[st_pos.at[0]])
            pltpu.sync_copy(st_idx, dsti.at[st_pos.at[0]])
            return carry
        lax.fori_loop(0, _SLICE // 128, pbody, jnp.int32(0))
        plsc.subcore_barrier()

    _radix_pass(sao, sai, sbo, sbi, lambda u: u & jnp.int32(0x7FF))
    _radix_pass(sbo, sbi, sao, sai,
                lambda u: _srl(u, 11) & jnp.int32(0x7FF))
    _radix_pass(sao, sai, sbo, sbi, lambda u: _srl(u, 22))

    pltpu.sync_copy(
        sbi.at[pl.ds(w * (N_OUT // _SW), N_OUT // _SW)],
        samples_out.at[pl.ds(w * (N_OUT // _SW), N_OUT // _SW)])


_BLK = 4096


def _tc_dyn_body(x_ref, u_ref, n_ref, at_ref, bt_ref, q_ref, r_ref,
                 newp_ref, selp_ref, logw_ref):
    x = x_ref[...]
    u = u_ref[...]
    newp_ref[...] = (
        jnp.dot(x, at_ref[...], preferred_element_type=jnp.float32,
                precision=lax.Precision.HIGHEST)
        + jnp.dot(u, bt_ref[...], preferred_element_type=jnp.float32,
                  precision=lax.Precision.HIGHEST)
        + SIG_DYN * n_ref[...])
    selp_ref[:, :DIM_X] = x
    selp_ref[:, DIM_X:] = u
    logw_ref[...] = -ALPHA * (
        jnp.sum(x * x * q_ref[...], axis=1, keepdims=True)
        + jnp.sum(u * u * r_ref[...], axis=1, keepdims=True))


def _tc_dynamics(sel_x, sel_u, noise, A, B, Q_diag, R_diag):
    grid = (N_OUT // _BLK,)
    return pl.pallas_call(
        _tc_dyn_body,
        grid=grid,
        in_specs=[
            pl.BlockSpec((_BLK, DIM_X), lambda i: (i, 0)),
            pl.BlockSpec((_BLK, DIM_U), lambda i: (i, 0)),
            pl.BlockSpec((_BLK, DIM_X), lambda i: (i, 0)),
            pl.BlockSpec((DIM_X, DIM_X), lambda i: (0, 0)),
            pl.BlockSpec((DIM_U, DIM_X), lambda i: (0, 0)),
            pl.BlockSpec((1, DIM_X), lambda i: (0, 0)),
            pl.BlockSpec((1, DIM_U), lambda i: (0, 0)),
        ],
        out_specs=[
            pl.BlockSpec((_BLK, DIM_X), lambda i: (i, 0)),
            pl.BlockSpec((_BLK, DIM_X + DIM_U), lambda i: (i, 0)),
            pl.BlockSpec((_BLK, 1), lambda i: (i, 0)),
        ],
        out_shape=[
            jax.ShapeDtypeStruct((N_OUT, DIM_X), jnp.float32),
            jax.ShapeDtypeStruct((N_OUT, DIM_X + DIM_U), jnp.float32),
            jax.ShapeDtypeStruct((N_OUT, 1), jnp.float32),
        ],
    )(sel_x, sel_u, noise, A.T, B.T, Q_diag[None, :], R_diag[None, :])


def kernel(particles, iteration, K, b, A, B, Q_diag, R_diag):
    key = jax.random.fold_in(jax.random.key(42), iteration)
    k1, k2, k3 = jax.random.split(key, 3)
    n_out = N_OUT
    mean_u = particles @ K.T + b
    mean_u_rep = jnp.repeat(mean_u, U_SAMPLES, axis=0)
    eps = jax.random.normal(k1, mean_u_rep.shape, dtype=jnp.float32)
    new_u = mean_u_rep + SIGMA_U * eps
    part_rep = jnp.repeat(particles, U_SAMPLES, axis=0)
    cost = (jnp.sum(part_rep * part_rep * Q_diag[None, :], axis=1)
            + jnp.sum(new_u * new_u * R_diag[None, :], axis=1))
    log_w = -ALPHA * cost
    logits = log_w - jax.scipy.special.logsumexp(log_w)
    u01 = jax.random.uniform(k2, logits.shape, dtype=jnp.float32)
    gumbel = -jnp.log(-jnp.log(u01 + 1e-20) + 1e-20)
    keys = jax.lax.stop_gradient(logits) + gumbel
    samples = _sc_topk_sort(keys.reshape(N_KEYS // 128, 128))
    dyn_noise = jax.random.normal(k3, (n_out, DIM_X), dtype=jnp.float32)

    samples2d = samples.reshape(N_OUT // 128, 128)
    p128 = particles.reshape(N_OUT * DIM_X // 128, 128)
    nu128 = new_u.reshape(NUM_P * DIM_U // 128, 128)
    sel_xf, sel_uf, anc2d = _sc_gather(samples2d, p128, nu128)
    sel_x = sel_xf.reshape(N_OUT, DIM_X)
    sel_u = sel_uf.reshape(N_OUT, DIM_U)
    ancestors = anc2d.reshape(N_OUT)
    new_particles, sel_particles, logw2d = _tc_dynamics(
        sel_x, sel_u, dyn_noise, A, B, Q_diag, R_diag)
    return (new_particles, sel_particles, logw2d.reshape(N_OUT), ancestors)
